# Initial kernel scaffold; baseline (speedup 1.0000x reference)
import jax
import jax.numpy as jnp
from jax.experimental import pallas as pl


def kernel(x, edge_index, W1, b1, W2, b2, Wp, bp):
    raise NotImplementedError("write your pallas kernel here")


# speedup vs baseline: 12.2867x; 12.2867x over previous
import functools

import jax
import jax.numpy as jnp
from jax import lax
from jax.experimental import pallas as pl
from jax.experimental.pallas import tpu as pltpu
from jax.experimental.pallas import tpu_sc as plsc

N = 10000
E = 320000
D = 128
NPAD = 10112
NC, NS = 2, 16
NW = NC * NS
EPW = E // NW
CHUNK = 80
NCHUNK = EPW // CHUNK
RPS = NPAD // NS
DEG_W = 128
BLK = 2000


_mesh = plsc.VectorSubcoreMesh(core_axis_name="c", subcore_axis_name="s")


@functools.partial(
    pl.kernel,
    out_type=jax.ShapeDtypeStruct((NC, NPAD, DEG_W), jnp.float32),
    mesh=_mesh,
    scratch_types=[
        pltpu.VMEM_SHARED((NPAD, DEG_W), jnp.float32),
        pltpu.VMEM((CHUNK,), jnp.int32),
        pltpu.VMEM((CHUNK, DEG_W), jnp.float32),
    ],
)
def _deg_kernel(dst_hbm, zeros_hbm, ones_hbm, out_hbm, acc, dst_v, ones_v):
    c = lax.axis_index("c")
    s = lax.axis_index("s")
    row0 = pl.multiple_of(s * RPS, 8)
    pltpu.sync_copy(zeros_hbm.at[pl.ds(row0, RPS)], acc.at[pl.ds(row0, RPS)])
    pltpu.sync_copy(ones_hbm, ones_v)
    plsc.subcore_barrier()
    base = (c * NS + s) * EPW

    def body(i, carry):
        off = base + i * CHUNK
        pltpu.sync_copy(dst_hbm.at[pl.ds(off, CHUNK)], dst_v)
        pltpu.sync_copy(ones_v, acc.at[dst_v], add=True)
        return carry

    lax.fori_loop(0, NCHUNK, body, 0)
    plsc.subcore_barrier()
    pltpu.sync_copy(acc.at[pl.ds(row0, RPS)], out_hbm.at[c, pl.ds(row0, RPS)])


@functools.partial(
    pl.kernel,
    out_type=jax.ShapeDtypeStruct((NC, NPAD, D), jnp.float32),
    mesh=_mesh,
    scratch_types=[
        pltpu.VMEM_SHARED((NPAD, D), jnp.float32),
        pltpu.VMEM((CHUNK,), jnp.int32),
        pltpu.VMEM((CHUNK,), jnp.int32),
        pltpu.VMEM((CHUNK, D), jnp.float32),
        pltpu.SemaphoreType.DMA,
    ],
)
def _edge_kernel(g_hbm, src_hbm, dst_hbm, zeros_hbm, out_hbm,
                 acc, src_v, dst_v, rows_v, sem):
    c = lax.axis_index("c")
    s = lax.axis_index("s")
    row0 = pl.multiple_of(s * RPS, 8)
    pltpu.sync_copy(zeros_hbm.at[pl.ds(row0, RPS)], acc.at[pl.ds(row0, RPS)])
    plsc.subcore_barrier()
    base = (c * NS + s) * EPW

    def body(i, carry):
        off = base + i * CHUNK
        pltpu.sync_copy(src_hbm.at[pl.ds(off, CHUNK)], src_v)
        pltpu.sync_copy(dst_hbm.at[pl.ds(off, CHUNK)], dst_v)
        pltpu.async_copy(g_hbm.at[src_v], rows_v, sem).wait()
        pltpu.sync_copy(rows_v, acc.at[dst_v], add=True)
        return carry

    lax.fori_loop(0, NCHUNK, body, 0)
    plsc.subcore_barrier()
    pltpu.sync_copy(acc.at[pl.ds(row0, RPS)], out_hbm.at[c, pl.ds(row0, RPS)])


def _g1_body(x_ref, w_ref, parts_ref, g_ref, dinv_ref):
    deg = parts_ref[0, :, 0:1] + parts_ref[1, :, 0:1] + 1.0
    dinv = lax.rsqrt(deg)
    h = jnp.dot(x_ref[...], w_ref[...], preferred_element_type=jnp.float32)
    g_ref[...] = h * dinv
    dinv_ref[...] = dinv


def _g1_call(x, W1, parts):
    return pl.pallas_call(
        _g1_body,
        grid=(N // BLK,),
        in_specs=[
            pl.BlockSpec((BLK, D), lambda i: (i, 0)),
            pl.BlockSpec((D, D), lambda i: (0, 0)),
            pl.BlockSpec((NC, BLK, DEG_W), lambda i: (0, i, 0)),
        ],
        out_specs=[
            pl.BlockSpec((BLK, D), lambda i: (i, 0)),
            pl.BlockSpec((BLK, 1), lambda i: (i, 0)),
        ],
        out_shape=[
            jax.ShapeDtypeStruct((N, D), jnp.float32),
            jax.ShapeDtypeStruct((N, 1), jnp.float32),
        ],
    )(x, W1, parts)


def _mid_body(sp_ref, g_ref, dinv_ref, b_ref, w_ref, out_ref):
    t = sp_ref[0] + sp_ref[1] + g_ref[...]
    t = jnp.maximum(t * dinv_ref[...] + b_ref[...], 0.0)
    out_ref[...] = (
        jnp.dot(t, w_ref[...], preferred_element_type=jnp.float32) * dinv_ref[...]
    )


def _mid_call(sp, g, dinv, b, W):
    return pl.pallas_call(
        _mid_body,
        grid=(N // BLK,),
        in_specs=[
            pl.BlockSpec((NC, BLK, D), lambda i: (0, i, 0)),
            pl.BlockSpec((BLK, D), lambda i: (i, 0)),
            pl.BlockSpec((BLK, 1), lambda i: (i, 0)),
            pl.BlockSpec((1, D), lambda i: (0, 0)),
            pl.BlockSpec((D, D), lambda i: (0, 0)),
        ],
        out_specs=pl.BlockSpec((BLK, D), lambda i: (i, 0)),
        out_shape=jax.ShapeDtypeStruct((N, D), jnp.float32),
    )(sp, g, dinv, b, W)


def _fin_body(sp_ref, g_ref, dinv_ref, b_ref, w_ref, bp_ref, out_ref):
    t = sp_ref[0] + sp_ref[1] + g_ref[...]
    t = jnp.maximum(t * dinv_ref[...] + b_ref[...], 0.0)
    out_ref[...] = (
        jnp.dot(t, w_ref[...], preferred_element_type=jnp.float32) + bp_ref[...]
    )


def _fin_call(sp, g, dinv, b, Wp, bp):
    n_out = Wp.shape[1]
    return pl.pallas_call(
        _fin_body,
        grid=(N // BLK,),
        in_specs=[
            pl.BlockSpec((NC, BLK, D), lambda i: (0, i, 0)),
            pl.BlockSpec((BLK, D), lambda i: (i, 0)),
            pl.BlockSpec((BLK, 1), lambda i: (i, 0)),
            pl.BlockSpec((1, D), lambda i: (0, 0)),
            pl.BlockSpec((D, n_out), lambda i: (0, 0)),
            pl.BlockSpec((1, n_out), lambda i: (0, 0)),
        ],
        out_specs=pl.BlockSpec((BLK, n_out), lambda i: (i, 0)),
        out_shape=jax.ShapeDtypeStruct((N, n_out), jnp.float32),
    )(sp, g, dinv, b, Wp, bp)


def kernel(x, edge_index, W1, b1, W2, b2, Wp, bp):
    src = edge_index[0]
    dst = edge_index[1]
    ones_chunk = jnp.ones((CHUNK, DEG_W), jnp.float32)
    zeros_d = jnp.zeros((NPAD, D), jnp.float32)

    deg_parts = _deg_kernel(dst, zeros_d, ones_chunk)
    g1, dinv = _g1_call(x, W1, deg_parts)
    sp1 = _edge_kernel(g1, src, dst, zeros_d)
    g2 = _mid_call(sp1, g1, dinv, b1.reshape(1, D), W2)
    sp2 = _edge_kernel(g2, src, dst, zeros_d)
    return _fin_call(sp2, g2, dinv, b2.reshape(1, D), Wp, bp.reshape(1, -1))


# speedup vs baseline: 32.8933x; 2.6772x over previous
import functools

import jax
import jax.numpy as jnp
from jax import lax
from jax.experimental import pallas as pl
from jax.experimental.pallas import tpu as pltpu
from jax.experimental.pallas import tpu_sc as plsc

N = 10000
E = 320000
D = 128
NPAD = 10112
NC, NS = 2, 16
NW = NC * NS
CHUNK = 128
NCHW = 80
EPAD = NW * NCHW * CHUNK
RPS = NPAD // NS
NBUF = 2
BLK = NPAD // 8
FBLK = 2000


_mesh = plsc.VectorSubcoreMesh(core_axis_name="c", subcore_axis_name="s")


@functools.partial(
    pl.kernel,
    out_type=jax.ShapeDtypeStruct((NC * NPAD,), jnp.float32),
    mesh=_mesh,
    scratch_types=[
        pltpu.VMEM_SHARED((NPAD,), jnp.float32),
        pltpu.VMEM((NCHW, CHUNK), jnp.int32),
        pltpu.VMEM((CHUNK,), jnp.float32),
        pltpu.VMEM((NPAD,), jnp.float32),
        pltpu.SemaphoreType.DMA,
    ],
)
def _deg_kernel(dstr_hbm, zeros1_hbm, ones1_hbm, out_hbm, acc, dsti, ones_v,
                bounce, sem):
    c = lax.axis_index("c")
    s = lax.axis_index("s")
    w = c * NS + s
    row0 = pl.multiple_of(s * RPS, 8)
    pltpu.sync_copy(dstr_hbm.at[w], dsti)
    pltpu.sync_copy(ones1_hbm, ones_v)
    pltpu.sync_copy(zeros1_hbm.at[pl.ds(row0, RPS)], bounce.at[pl.ds(0, RPS)])
    pltpu.sync_copy(bounce.at[pl.ds(0, RPS)], acc.at[pl.ds(row0, RPS)])
    plsc.subcore_barrier()

    def fire(j, carry):
        pltpu.async_copy(ones_v, acc.at[dsti.at[j]], sem, add=True)
        return carry

    lax.fori_loop(0, NCHW, fire, 0)

    def drain(j, carry):
        pltpu.make_async_copy(ones_v, acc.at[dsti.at[0]], sem).wait()
        return carry

    lax.fori_loop(0, NCHW, drain, 0)
    plsc.subcore_barrier()

    @pl.when(s == 0)
    def _():
        out0 = pl.multiple_of(c * NPAD, 8)
        pltpu.sync_copy(acc, bounce)
        pltpu.sync_copy(bounce, out_hbm.at[pl.ds(out0, NPAD)])


@functools.partial(
    pl.kernel,
    out_type=jax.ShapeDtypeStruct((NC, NPAD, D), jnp.float32),
    mesh=_mesh,
    scratch_types=[
        pltpu.VMEM_SHARED((NPAD, D), jnp.float32),
        pltpu.VMEM((NCHW, CHUNK), jnp.int32),
        pltpu.VMEM((CHUNK,), jnp.int32),
        pltpu.VMEM((CHUNK,), jnp.int32),
        pltpu.VMEM((CHUNK, D), jnp.float32),
        pltpu.VMEM((CHUNK, D), jnp.float32),
        pltpu.SemaphoreType.DMA,
        pltpu.SemaphoreType.DMA,
        pltpu.SemaphoreType.DMA,
        pltpu.SemaphoreType.DMA,
        pltpu.SemaphoreType.DMA,
        pltpu.SemaphoreType.DMA,
    ],
)
def _edge_kernel(g_hbm, srcr_hbm, dstf_hbm, zeros_hbm, out_hbm,
                 acc, srci, dstv0, dstv1, rows0, rows1,
                 gsem0, gsem1, ssem0, ssem1, isem0, isem1):
    c = lax.axis_index("c")
    s = lax.axis_index("s")
    w = c * NS + s
    row0 = pl.multiple_of(s * RPS, 8)
    pltpu.sync_copy(srcr_hbm.at[w], srci)
    pltpu.sync_copy(zeros_hbm.at[pl.ds(row0, RPS)], acc.at[pl.ds(row0, RPS)])
    plsc.subcore_barrier()

    dstv = (dstv0, dstv1)
    rows = (rows0, rows1)
    gsem = (gsem0, gsem1)
    ssem = (ssem0, ssem1)
    isem = (isem0, isem1)
    ebase = w * (NCHW * CHUNK)

    def idx_src(j):
        return dstf_hbm.at[pl.ds(pl.multiple_of(ebase + j * CHUNK, 8), CHUNK)]

    def start_idx(j, b):
        pltpu.async_copy(idx_src(j), dstv[b], isem[b])

    def wait_idx(j, b):
        pltpu.make_async_copy(idx_src(j), dstv[b], isem[b]).wait()

    def start_gather(j, b):
        pltpu.async_copy(g_hbm.at[srci.at[j]], rows[b], gsem[b])

    def wait_gather(j, b):
        pltpu.make_async_copy(g_hbm.at[srci.at[j]], rows[b], gsem[b]).wait()

    def start_scatter(j, b):
        pltpu.async_copy(rows[b], acc.at[dstv[b]], ssem[b], add=True)

    def wait_scatter(j, b):
        pltpu.make_async_copy(rows[b], acc.at[dstv[b]], ssem[b]).wait()

    for b in range(NBUF):
        start_idx(b, b)
        start_gather(b, b)

    def body(k, carry):
        g0 = k * NBUF
        for b in range(NBUF):
            j = g0 + b
            wait_gather(j, b)
            wait_idx(j, b)
            start_scatter(j, b)
            wait_scatter(j, b)
            start_idx(j + NBUF, b)
            start_gather(j + NBUF, b)
        return carry

    lax.fori_loop(0, NCHW // NBUF - 1, body, 0)

    for b in range(NBUF):
        j = NCHW - NBUF + b
        wait_gather(j, b)
        wait_idx(j, b)
        start_scatter(j, b)
        wait_scatter(j, b)

    plsc.subcore_barrier()
    pltpu.sync_copy(acc.at[pl.ds(row0, RPS)], out_hbm.at[c, pl.ds(row0, RPS)])


def _g1_body(x_ref, w_ref, degt_ref, g_ref, dinv_ref):
    deg = degt_ref[:, 0:1] + degt_ref[:, 1:2] + 1.0
    dinv = lax.rsqrt(deg)
    h = jnp.dot(x_ref[...], w_ref[...], preferred_element_type=jnp.float32)
    g_ref[...] = h * dinv
    dinv_ref[...] = dinv


def _g1_call(x, W1, degt):
    return pl.pallas_call(
        _g1_body,
        grid=(NPAD // BLK,),
        in_specs=[
            pl.BlockSpec((BLK, D), lambda i: (i, 0)),
            pl.BlockSpec((D, D), lambda i: (0, 0)),
            pl.BlockSpec((BLK, NC), lambda i: (i, 0)),
        ],
        out_specs=[
            pl.BlockSpec((BLK, D), lambda i: (i, 0)),
            pl.BlockSpec((BLK, 1), lambda i: (i, 0)),
        ],
        out_shape=[
            jax.ShapeDtypeStruct((NPAD, D), jnp.float32),
            jax.ShapeDtypeStruct((NPAD, 1), jnp.float32),
        ],
    )(x, W1, degt)


def _mid_body(sp_ref, g_ref, dinv_ref, b_ref, w_ref, out_ref):
    t = sp_ref[0] + sp_ref[1] + g_ref[...]
    t = jnp.maximum(t * dinv_ref[...] + b_ref[...], 0.0)
    out_ref[...] = (
        jnp.dot(t, w_ref[...], preferred_element_type=jnp.float32) * dinv_ref[...]
    )


def _mid_call(sp, g, dinv, b, W):
    return pl.pallas_call(
        _mid_body,
        grid=(NPAD // BLK,),
        in_specs=[
            pl.BlockSpec((NC, BLK, D), lambda i: (0, i, 0)),
            pl.BlockSpec((BLK, D), lambda i: (i, 0)),
            pl.BlockSpec((BLK, 1), lambda i: (i, 0)),
            pl.BlockSpec((1, D), lambda i: (0, 0)),
            pl.BlockSpec((D, D), lambda i: (0, 0)),
        ],
        out_specs=pl.BlockSpec((BLK, D), lambda i: (i, 0)),
        out_shape=jax.ShapeDtypeStruct((NPAD, D), jnp.float32),
    )(sp, g, dinv, b, W)


def _fin_body(sp_ref, g_ref, dinv_ref, b_ref, w_ref, bp_ref, out_ref):
    t = sp_ref[0] + sp_ref[1] + g_ref[...]
    t = jnp.maximum(t * dinv_ref[...] + b_ref[...], 0.0)
    out_ref[...] = (
        jnp.dot(t, w_ref[...], preferred_element_type=jnp.float32) + bp_ref[...]
    )


def _fin_call(sp, g, dinv, b, Wp, bp):
    n_out = Wp.shape[1]
    return pl.pallas_call(
        _fin_body,
        grid=(N // FBLK,),
        in_specs=[
            pl.BlockSpec((NC, FBLK, D), lambda i: (0, i, 0)),
            pl.BlockSpec((FBLK, D), lambda i: (i, 0)),
            pl.BlockSpec((FBLK, 1), lambda i: (i, 0)),
            pl.BlockSpec((1, D), lambda i: (0, 0)),
            pl.BlockSpec((D, n_out), lambda i: (0, 0)),
            pl.BlockSpec((1, n_out), lambda i: (0, 0)),
        ],
        out_specs=pl.BlockSpec((FBLK, n_out), lambda i: (i, 0)),
        out_shape=jax.ShapeDtypeStruct((N, n_out), jnp.float32),
    )(sp, g, dinv, b, Wp, bp)


def kernel(x, edge_index, W1, b1, W2, b2, Wp, bp):
    src = edge_index[0]
    dst = edge_index[1]
    pad_idx = N + (jnp.arange(EPAD - E, dtype=jnp.int32) % (NPAD - N))
    src_p = jnp.concatenate([src, pad_idx]).reshape(NW, NCHW, CHUNK)
    dst_p = jnp.concatenate([dst, pad_idx]).reshape(NW, NCHW, CHUNK)
    x_p = jnp.pad(x, ((0, NPAD - N), (0, 0)))
    zeros1 = jnp.zeros((NPAD,), jnp.float32)
    ones1 = jnp.ones((CHUNK,), jnp.float32)
    zeros_d = jnp.zeros((NPAD, D), jnp.float32)

    dst_f = dst_p.reshape(EPAD)
    deg_flat = _deg_kernel(dst_p, zeros1, ones1)
    degt = deg_flat.reshape(NC, NPAD).T
    g1, dinv = _g1_call(x_p, W1, degt)
    sp1 = _edge_kernel(g1, src_p, dst_f, zeros_d)
    g2 = _mid_call(sp1, g1, dinv, b1.reshape(1, D), W2)
    sp2 = _edge_kernel(g2, src_p, dst_f, zeros_d)
    return _fin_call(sp2, g2, dinv, b2.reshape(1, D), Wp, bp.reshape(1, -1))
